# Initial kernel scaffold; baseline (speedup 1.0000x reference)
import jax
import jax.numpy as jnp
from jax.experimental import pallas as pl


def kernel(inputs, targets, features):
    raise NotImplementedError("write your pallas kernel here")


# speedup vs baseline: 4.0063x; 4.0063x over previous
import functools

import jax
import jax.numpy as jnp
from jax import lax
from jax.experimental import pallas as pl
from jax.experimental.pallas import tpu as pltpu
from jax.experimental.pallas import tpu_sc as plsc

_TEMP = 0.05
_EPS = 1e-12


def _sc_gather(table, idx, B, D):
    info = plsc.get_sparse_core_info()
    nw = info.num_cores * info.num_subcores
    b_per_w = B // nw
    mesh = plsc.VectorSubcoreMesh(core_axis_name="c", subcore_axis_name="s")

    @functools.partial(
        pl.kernel,
        mesh=mesh,
        out_type=jax.ShapeDtypeStruct((B, D), jnp.float32),
        compiler_params=pltpu.CompilerParams(use_tc_tiling_on_sc=False),
        scratch_types=[
            pltpu.VMEM((b_per_w,), jnp.int32),
            pltpu.VMEM((b_per_w, D), jnp.float32),
            pltpu.SemaphoreType.DMA,
        ],
    )
    def gather_kernel(table_hbm, idx_hbm, out_hbm, idx_v, rows_v, sem):
        wid = lax.axis_index("s") * info.num_cores + lax.axis_index("c")
        base = wid * b_per_w
        pltpu.sync_copy(idx_hbm.at[pl.ds(base, b_per_w)], idx_v)
        pltpu.async_copy(table_hbm.at[idx_v], rows_v, sem).wait()
        pltpu.sync_copy(rows_v, out_hbm.at[pl.ds(base, b_per_w)])

    return gather_kernel(table, idx)


def _tc_body(inv_b, x_ref, tgt_ref, feat_ref, out_ref):
    i = pl.program_id(0)
    x = x_ref[...]
    norm = jnp.sqrt(jnp.sum(x * x, axis=1, keepdims=True))
    xn = x / (norm + _EPS)
    logits = jnp.dot(xn, feat_ref[...], preferred_element_type=jnp.float32)
    se = jnp.sum(jnp.exp(logits * (1.0 / _TEMP)), axis=1, keepdims=True)
    lse = jnp.log(se)
    tgt = jnp.sum(xn * tgt_ref[...], axis=1, keepdims=True) * (1.0 / _TEMP)
    partial = jnp.sum(lse - tgt, keepdims=True) * inv_b

    @pl.when(i == 0)
    def _init():
        out_ref[...] = jnp.zeros_like(out_ref)

    out_ref[...] += partial


def _tc_loss(x, tgt_feats, feat_t, BT):
    B, D = x.shape
    K = feat_t.shape[1]
    grid = (B // BT,)
    return pl.pallas_call(
        functools.partial(_tc_body, 1.0 / B),
        grid=grid,
        in_specs=[
            pl.BlockSpec((BT, D), lambda i: (i, 0)),
            pl.BlockSpec((BT, D), lambda i: (i, 0)),
            pl.BlockSpec((D, K), lambda i: (0, 0)),
        ],
        out_specs=pl.BlockSpec((1, 1), lambda i: (0, 0)),
        out_shape=jax.ShapeDtypeStruct((1, 1), jnp.float32),
    )(x, tgt_feats, feat_t)


def kernel(inputs, targets, features):
    B, D = inputs.shape
    tgt_feats = _sc_gather(features, targets.astype(jnp.int32), B, D)
    out = _tc_loss(inputs, tgt_feats, features.T, 512)
    return out[0, 0]
